# Initial kernel scaffold; baseline (speedup 1.0000x reference)
import jax
import jax.numpy as jnp
from jax.experimental import pallas as pl


def kernel(A_1_featurs, W_fe, b_fe, weights_pool, bias_pool, factor_embeddings, Wq, bq, Wk, bk, W_sub, b_sub, W_agg, b_agg, W_up, b_up, W_dec, b_dec, s1, s2, a1, nbr2, nbr3):
    raise NotImplementedError("write your pallas kernel here")


# speedup vs baseline: 3.9303x; 3.9303x over previous
import jax
import jax.numpy as jnp
from jax.experimental import pallas as pl

B, T, F = 4, 7, 8
HID, EMB = 32, 16
D = F * HID
N, N2, CS, K2 = 1024, 64, 16, 8
RT = 256
NT = N // RT
CHUNK = 128

_INTERPRET = False


def _iota_eq(shape, dim0, dim1, div0, div1, dtype=jnp.float32):
    i0 = jax.lax.broadcasted_iota(jnp.int32, shape, dim0) // div0
    i1 = jax.lax.broadcasted_iota(jnp.int32, shape, dim1) // div1
    return (i0 == i1).astype(dtype)


def _prep_kernel(xn_ref, wfe_ref, bfe_ref, fe_ref, wp0_ref, wp1_ref,
                 bpool_ref, wq_ref, bq_ref, wk_ref, bk_ref,
                 x1_ref, q_ref, k_ref, a2_ref):
    f32 = jnp.float32
    xn = xn_ref[0]

    fe = fe_ref[...]
    g = jnp.maximum(jnp.dot(fe, fe.T, preferred_element_type=f32, precision=jax.lax.Precision.HIGHEST), 0.0)
    g = g - jnp.max(g, axis=1, keepdims=True)
    eg = jnp.exp(g)
    supports = eg / jnp.sum(eg, axis=1, keepdims=True)
    sup_t = supports.T

    e1 = _iota_eq((D, F), 0, 1, HID, 1)
    e2 = _iota_eq((EMB, EMB * HID), 1, 0, HID, 1)
    r32 = jax.lax.broadcasted_iota(jnp.int32, (D, EMB * HID), 0) % HID
    c32 = jax.lax.broadcasted_iota(jnp.int32, (D, EMB * HID), 1) % HID
    diag = (r32 == c32).astype(f32)
    kron = jnp.dot(jnp.dot(e1, fe, preferred_element_type=f32, precision=jax.lax.Precision.HIGHEST), e2,
                   preferred_element_type=f32, precision=jax.lax.Precision.HIGHEST) * diag
    w0all = jnp.dot(kron, wp0_ref[...], preferred_element_type=f32, precision=jax.lax.Precision.HIGHEST)
    w1all = jnp.dot(kron, wp1_ref[...], preferred_element_type=f32, precision=jax.lax.Precision.HIGHEST)

    bias8 = jnp.dot(fe, bpool_ref[...], preferred_element_type=f32, precision=jax.lax.Precision.HIGHEST)
    bias_flat = jnp.concatenate([bias8[f:f + 1, :] for f in range(F)], axis=1)

    cols = []
    for f in range(F):
        w0t = jnp.concatenate([w0all[f * HID:(f + 1) * HID, :]] * F, axis=0)
        w1t = jnp.concatenate([w1all[f * HID:(f + 1) * HID, :]] * F, axis=0)
        scale = jnp.dot(e1, sup_t[:, f:f + 1], preferred_element_type=f32, precision=jax.lax.Precision.HIGHEST)
        cols.append(scale * w1t + e1[:, f:f + 1] * w0t)
    wcomb = jnp.concatenate(cols, axis=1)

    wfe = wfe_ref[...]
    fcols = []
    for f in range(F):
        parts = []
        if f > 0:
            parts.append(jnp.zeros((T * f, HID), f32))
        parts.append(wfe)
        if f < F - 1:
            parts.append(jnp.zeros((T * (F - 1 - f), HID), f32))
        fcols.append(jnp.concatenate(parts, axis=0))
    wblk = jnp.concatenate(fcols, axis=1)

    wfused = jnp.dot(wblk, wcomb, preferred_element_type=f32, precision=jax.lax.Precision.HIGHEST)
    bfe_rep = jnp.concatenate([bfe_ref[...]] * F, axis=1)
    bias_row = jnp.dot(bfe_rep, wcomb, preferred_element_type=f32, precision=jax.lax.Precision.HIGHEST) + bias_flat

    x1 = jnp.dot(xn, wfused, preferred_element_type=f32, precision=jax.lax.Precision.HIGHEST) + bias_row

    x1_ref[0] = x1
    q_ref[0] = (jnp.dot(x1, wq_ref[...], preferred_element_type=f32, precision=jax.lax.Precision.HIGHEST)
                + bq_ref[...]) * (1.0 / 16.0)
    k_ref[0] = jnp.dot(x1, wk_ref[...], preferred_element_type=f32, precision=jax.lax.Precision.HIGHEST) + bk_ref[...]
    poolm = _iota_eq((N2, N), 0, 1, 1, CS)
    a2_ref[0] = jnp.dot(poolm, x1, preferred_element_type=f32, precision=jax.lax.Precision.HIGHEST)


def _attn_kernel(q_ref, k_ref, out_ref):
    f32 = jnp.float32
    b = pl.program_id(1)
    q = q_ref[0]
    k = k_ref[0]
    s = jax.lax.dot_general(q, k, (((1,), (1,)), ((), ())),
                            preferred_element_type=f32, precision=jax.lax.Precision.HIGHEST)
    m = jnp.max(s, axis=1, keepdims=True)
    e = jnp.exp(s - m)
    rs = jnp.sum(e, axis=1, keepdims=True)
    cmask = _iota_eq((N, N2), 0, 1, CS, 1)
    pooled = jnp.dot(e, cmask, preferred_element_type=f32, precision=jax.lax.Precision.HIGHEST) / rs
    rmask = _iota_eq((RT, RT // CS), 0, 1, CS, 1)
    part = jax.lax.dot_general(rmask, pooled, (((0,), (0,)), ((), ())),
                               preferred_element_type=f32, precision=jax.lax.Precision.HIGHEST) * 0.25

    @pl.when(b == 0)
    def _():
        out_ref[...] = part

    @pl.when(b > 0)
    def _():
        out_ref[...] = out_ref[...] + part


def _mp_kernel(x1_ref, a2_ref, a2dyn_ref, a1_ref, nbr2_ref,
               wsub_ref, bsub_ref, wagg_ref, bagg_ref,
               wup_ref, bup_ref, wdec_ref, bdec_ref, out_ref):
    f32 = jnp.float32
    x1 = x1_ref[0]

    dmask = _iota_eq((CHUNK, CHUNK), 0, 1, CS, CS)
    zs = []
    for c in range(N // CHUNK):
        g = a1_ref[c * CHUNK:(c + 1) * CHUNK, c * CHUNK:(c + 1) * CHUNK] * dmask
        xc = x1[c * CHUNK:(c + 1) * CHUNK, :]
        zs.append(jax.lax.dot_general(g, xc, (((0,), (0,)), ((), ())),
                                      preferred_element_type=f32, precision=jax.lax.Precision.HIGHEST))
    z = jnp.concatenate(zs, axis=0)
    m1 = jnp.maximum(jnp.dot(z, wsub_ref[...], preferred_element_type=f32, precision=jax.lax.Precision.HIGHEST)
                     + bsub_ref[...], 0.0)

    wa = wagg_ref[...]
    s_node = jnp.dot(m1, wa[0:D, :], preferred_element_type=f32, precision=jax.lax.Precision.HIGHEST)

    nbr = nbr2_ref[...]
    iota2 = jax.lax.broadcasted_iota(jnp.int32, (N2, N2), 1)
    cnt = jnp.zeros((N2, N2), f32)
    for kk in range(K2):
        cnt = cnt + (nbr[:, kk:kk + 1] == iota2).astype(f32)
    r = a2dyn_ref[...] * cnt
    m2 = jnp.dot(r, a2_ref[0], preferred_element_type=f32, precision=jax.lax.Precision.HIGHEST)
    cvec = jnp.dot(m2, wa[D:2 * D, :] + wa[2 * D:3 * D, :],
                   preferred_element_type=f32, precision=jax.lax.Precision.HIGHEST)
    rep = _iota_eq((N, N2), 0, 1, CS, 1)
    crep = jnp.dot(rep, cvec, preferred_element_type=f32, precision=jax.lax.Precision.HIGHEST)

    agg = jnp.maximum(s_node + crep + bagg_ref[0, 0], 0.0)
    upd = jnp.maximum(jnp.dot(x1 + agg, wup_ref[...], preferred_element_type=f32, precision=jax.lax.Precision.HIGHEST)
                      + bup_ref[...], 0.0)
    out_ref[0] = jnp.dot(upd, wdec_ref[...], preferred_element_type=f32, precision=jax.lax.Precision.HIGHEST) + bdec_ref[...]


def kernel(A_1_featurs, W_fe, b_fe, weights_pool, bias_pool, factor_embeddings,
           Wq, bq, Wk, bk, W_sub, b_sub, W_agg, b_agg, W_up, b_up,
           W_dec, b_dec, s1, s2, a1, nbr2, nbr3):
    f32 = jnp.float32
    xn = jnp.transpose(A_1_featurs.reshape(B, T, N, F), (0, 2, 3, 1)).reshape(B, N, F * T)
    wp0 = weights_pool[:, 0].reshape(EMB * HID, HID)
    wp1 = weights_pool[:, 1].reshape(EMB * HID, HID)
    b_fe2 = b_fe.reshape(1, HID)
    bq2 = bq.reshape(1, D)
    bk2 = bk.reshape(1, D)
    bsub2 = b_sub.reshape(1, D)
    bagg2 = b_agg.reshape(1, 1)
    bup2 = b_up.reshape(1, D)
    bdec2 = b_dec.reshape(1, 5)

    x1, q, k, a2 = pl.pallas_call(
        _prep_kernel,
        grid=(B,),
        in_specs=[
            pl.BlockSpec((1, N, F * T), lambda b: (b, 0, 0)),
            pl.BlockSpec((T, HID), lambda b: (0, 0)),
            pl.BlockSpec((1, HID), lambda b: (0, 0)),
            pl.BlockSpec((F, EMB), lambda b: (0, 0)),
            pl.BlockSpec((EMB * HID, HID), lambda b: (0, 0)),
            pl.BlockSpec((EMB * HID, HID), lambda b: (0, 0)),
            pl.BlockSpec((EMB, HID), lambda b: (0, 0)),
            pl.BlockSpec((D, D), lambda b: (0, 0)),
            pl.BlockSpec((1, D), lambda b: (0, 0)),
            pl.BlockSpec((D, D), lambda b: (0, 0)),
            pl.BlockSpec((1, D), lambda b: (0, 0)),
        ],
        out_specs=[
            pl.BlockSpec((1, N, D), lambda b: (b, 0, 0)),
            pl.BlockSpec((1, N, D), lambda b: (b, 0, 0)),
            pl.BlockSpec((1, N, D), lambda b: (b, 0, 0)),
            pl.BlockSpec((1, N2, D), lambda b: (b, 0, 0)),
        ],
        out_shape=[
            jax.ShapeDtypeStruct((B, N, D), f32),
            jax.ShapeDtypeStruct((B, N, D), f32),
            jax.ShapeDtypeStruct((B, N, D), f32),
            jax.ShapeDtypeStruct((B, N2, D), f32),
        ],
        interpret=_INTERPRET,
    )(xn, W_fe, b_fe2, factor_embeddings, wp0, wp1, bias_pool,
      Wq, bq2, Wk, bk2)

    a2_dyn = pl.pallas_call(
        _attn_kernel,
        grid=(NT, B),
        in_specs=[
            pl.BlockSpec((1, RT, D), lambda t, b: (b, t, 0)),
            pl.BlockSpec((1, N, D), lambda t, b: (b, 0, 0)),
        ],
        out_specs=pl.BlockSpec((RT // CS, N2), lambda t, b: (t, 0)),
        out_shape=jax.ShapeDtypeStruct((N2, N2), f32),
        interpret=_INTERPRET,
    )(q, k)

    out = pl.pallas_call(
        _mp_kernel,
        grid=(B,),
        in_specs=[
            pl.BlockSpec((1, N, D), lambda b: (b, 0, 0)),
            pl.BlockSpec((1, N2, D), lambda b: (b, 0, 0)),
            pl.BlockSpec((N2, N2), lambda b: (0, 0)),
            pl.BlockSpec((N, N), lambda b: (0, 0)),
            pl.BlockSpec((N2, K2), lambda b: (0, 0)),
            pl.BlockSpec((D, D), lambda b: (0, 0)),
            pl.BlockSpec((1, D), lambda b: (0, 0)),
            pl.BlockSpec((3 * D, 1), lambda b: (0, 0)),
            pl.BlockSpec((1, 1), lambda b: (0, 0)),
            pl.BlockSpec((D, D), lambda b: (0, 0)),
            pl.BlockSpec((1, D), lambda b: (0, 0)),
            pl.BlockSpec((D, 5), lambda b: (0, 0)),
            pl.BlockSpec((1, 5), lambda b: (0, 0)),
        ],
        out_specs=pl.BlockSpec((1, N, 5), lambda b: (b, 0, 0)),
        out_shape=jax.ShapeDtypeStruct((B, N, 5), f32),
        interpret=_INTERPRET,
    )(x1, a2, a2_dyn, a1, nbr2,
      W_sub, bsub2, W_agg, bagg2, W_up, bup2, W_dec, bdec2)

    return out


# speedup vs baseline: 5.4619x; 1.3897x over previous
import jax
import jax.numpy as jnp
from jax.experimental import pallas as pl

B, T, F = 4, 7, 8
HID, EMB = 32, 16
D = F * HID
N, N2, CS, K2 = 1024, 64, 16, 8
RT = 256
NT = N // RT
CHUNK = 128

_INTERPRET = False
_HI = jax.lax.Precision.HIGHEST


def _dot(a, b):
    return jnp.dot(a, b, preferred_element_type=jnp.float32, precision=_HI)


def _dott(a, b):
    return jax.lax.dot_general(a, b, (((0,), (0,)), ((), ())),
                               preferred_element_type=jnp.float32, precision=_HI)


def _dotn(a, b):
    return jax.lax.dot_general(a, b, (((1,), (1,)), ((), ())),
                               preferred_element_type=jnp.float32, precision=_HI)


def _iota_eq(shape, dim0, dim1, div0, div1):
    i0 = jax.lax.broadcasted_iota(jnp.int32, shape, dim0) // div0
    i1 = jax.lax.broadcasted_iota(jnp.int32, shape, dim1) // div1
    return (i0 == i1).astype(jnp.float32)


def _mega_kernel(xn_ref, wfe_ref, bfe_ref, fe_ref, wp0_ref, wp1_ref,
                 bpool_ref, wq_ref, bq_ref, wk_ref, bk_ref,
                 a1_ref, nbr2_ref,
                 wsub_ref, bsub_ref, wagg_ref, bagg_ref,
                 wup_ref, bup_ref, wdec_ref, bdec_ref, out_ref):
    f32 = jnp.float32

    fe = fe_ref[...]
    g = jnp.maximum(_dot(fe, fe.T), 0.0)
    g = g - jnp.max(g, axis=1, keepdims=True)
    eg = jnp.exp(g)
    supports = eg / jnp.sum(eg, axis=1, keepdims=True)
    sup_t = supports.T

    e1 = _iota_eq((D, F), 0, 1, HID, 1)
    e2 = _iota_eq((EMB, EMB * HID), 1, 0, HID, 1)
    r32 = jax.lax.broadcasted_iota(jnp.int32, (D, EMB * HID), 0) % HID
    c32 = jax.lax.broadcasted_iota(jnp.int32, (D, EMB * HID), 1) % HID
    diag = (r32 == c32).astype(f32)
    kron = _dot(_dot(e1, fe), e2) * diag
    w0all = _dot(kron, wp0_ref[...])
    w1all = _dot(kron, wp1_ref[...])

    bias8 = _dot(fe, bpool_ref[...])
    bias_flat = jnp.concatenate([bias8[f:f + 1, :] for f in range(F)], axis=1)

    cols = []
    for f in range(F):
        w0t = jnp.concatenate([w0all[f * HID:(f + 1) * HID, :]] * F, axis=0)
        w1t = jnp.concatenate([w1all[f * HID:(f + 1) * HID, :]] * F, axis=0)
        scale = _dot(e1, sup_t[:, f:f + 1])
        cols.append(scale * w1t + e1[:, f:f + 1] * w0t)
    wcomb = jnp.concatenate(cols, axis=1)

    wfe = wfe_ref[...]
    fcols = []
    for f in range(F):
        parts = []
        if f > 0:
            parts.append(jnp.zeros((T * f, HID), f32))
        parts.append(wfe)
        if f < F - 1:
            parts.append(jnp.zeros((T * (F - 1 - f), HID), f32))
        fcols.append(jnp.concatenate(parts, axis=0))
    wblk = jnp.concatenate(fcols, axis=1)

    wfused = _dot(wblk, wcomb)
    bfe_rep = jnp.concatenate([bfe_ref[...]] * F, axis=1)
    bias_row = _dot(bfe_rep, wcomb) + bias_flat

    cpool = _iota_eq((N, N2), 0, 1, CS, 1)
    rmask = _iota_eq((RT, RT // CS), 0, 1, CS, 1)

    x1s, a2s, dyn_parts = [], [], []
    for b in range(B):
        x1_b = _dot(xn_ref[b], wfused) + bias_row
        q_b = (_dot(x1_b, wq_ref[...]) + bq_ref[...]) * (1.0 / 16.0)
        k_b = _dot(x1_b, wk_ref[...]) + bk_ref[...]
        x1s.append(x1_b)
        a2s.append(_dott(cpool, x1_b))
        rows = []
        for t in range(NT):
            s = _dotn(q_b[t * RT:(t + 1) * RT, :], k_b)
            m = jnp.max(s, axis=1, keepdims=True)
            e = jnp.exp(s - m)
            rs = jnp.sum(e, axis=1, keepdims=True)
            p = e * (1.0 / rs)
            rp = _dott(rmask, p)
            rows.append(_dot(rp, cpool))
        dyn_parts.append(jnp.concatenate(rows, axis=0))
    a2_dyn = (dyn_parts[0] + dyn_parts[1] + dyn_parts[2] + dyn_parts[3]) * 0.25

    nbr = nbr2_ref[...]
    iota2 = jax.lax.broadcasted_iota(jnp.int32, (N2, N2), 1)
    cnt = jnp.zeros((N2, N2), f32)
    for kk in range(K2):
        cnt = cnt + (nbr[:, kk:kk + 1] == iota2).astype(f32)
    r = a2_dyn * cnt

    wa = wagg_ref[...]
    wa23 = wa[D:2 * D, :] + wa[2 * D:3 * D, :]
    dmask = _iota_eq((CHUNK, CHUNK), 0, 1, CS, CS)
    for b in range(B):
        x1_b = x1s[b]
        zs = []
        for c in range(N // CHUNK):
            gc = a1_ref[c * CHUNK:(c + 1) * CHUNK, c * CHUNK:(c + 1) * CHUNK] * dmask
            zs.append(_dott(gc, x1_b[c * CHUNK:(c + 1) * CHUNK, :]))
        z = jnp.concatenate(zs, axis=0)
        m1 = jnp.maximum(_dot(z, wsub_ref[...]) + bsub_ref[...], 0.0)
        s_node = _dot(m1, wa[0:D, :])
        m2 = _dot(r, a2s[b])
        cvec = _dot(m2, wa23)
        crep = _dot(cpool, cvec)
        agg = jnp.maximum(s_node + crep + bagg_ref[0, 0], 0.0)
        upd = jnp.maximum(_dot(x1_b + agg, wup_ref[...]) + bup_ref[...], 0.0)
        out_ref[b] = _dot(upd, wdec_ref[...]) + bdec_ref[...]


def kernel(A_1_featurs, W_fe, b_fe, weights_pool, bias_pool, factor_embeddings,
           Wq, bq, Wk, bk, W_sub, b_sub, W_agg, b_agg, W_up, b_up,
           W_dec, b_dec, s1, s2, a1, nbr2, nbr3):
    f32 = jnp.float32
    xn = jnp.transpose(A_1_featurs.reshape(B, T, N, F), (0, 2, 3, 1)).reshape(B, N, F * T)
    wp0 = weights_pool[:, 0].reshape(EMB * HID, HID)
    wp1 = weights_pool[:, 1].reshape(EMB * HID, HID)
    args = (xn, W_fe, b_fe.reshape(1, HID), factor_embeddings, wp0, wp1,
            bias_pool, Wq, bq.reshape(1, D), Wk, bk.reshape(1, D),
            a1, nbr2, W_sub, b_sub.reshape(1, D), W_agg, b_agg.reshape(1, 1),
            W_up, b_up.reshape(1, D), W_dec, b_dec.reshape(1, 5))
    out = pl.pallas_call(
        _mega_kernel,
        grid=(1,),
        in_specs=[pl.BlockSpec(a.shape, lambda i, nd=a.ndim: (0,) * nd)
                  for a in args],
        out_specs=pl.BlockSpec((B, N, 5), lambda i: (0, 0, 0)),
        out_shape=jax.ShapeDtypeStruct((B, N, 5), f32),
        interpret=_INTERPRET,
    )(*args)
    return out
